# Initial kernel scaffold; baseline (speedup 1.0000x reference)
import jax
import jax.numpy as jnp
from jax.experimental import pallas as pl


def kernel(hm, wh, hps, reg, hm_hp, hp_offset):
    raise NotImplementedError("write your pallas kernel here")


# speedup vs baseline: 7.6379x; 7.6379x over previous
import jax
import jax.numpy as jnp
from jax.experimental import pallas as pl
from jax.experimental.pallas import tpu as pltpu

_K = 100
_J = 17
_H = 128
_W = 128
_NMAPS = 1 + _J
_C = 38
_THRESH = 0.1


def _decode_kernel(heat_ref, featT_ref, hpT_ref, out_ref, X):
    f32 = jnp.float32
    ninf = f32(-jnp.inf)

    x = jax.nn.sigmoid(heat_ref[...])
    negc = jnp.full((_NMAPS, _H, 1), ninf, f32)
    xl = jnp.concatenate([x[:, :, 1:], negc], axis=2)
    xr = jnp.concatenate([negc, x[:, :, :-1]], axis=2)
    m1 = jnp.maximum(jnp.maximum(xl, xr), x)
    negr = jnp.full((_NMAPS, 1, _W), ninf, f32)
    mu = jnp.concatenate([m1[:, 1:, :], negr], axis=1)
    md = jnp.concatenate([negr, m1[:, :-1, :]], axis=1)
    hmax = jnp.maximum(jnp.maximum(mu, md), m1)
    X[...] = jnp.where(hmax == x, x, 0.0)

    iota2 = (jax.lax.broadcasted_iota(jnp.int32, (1, _H, _W), 1) * _W
             + jax.lax.broadcasted_iota(jnp.int32, (1, _H, _W), 2))
    lane = jax.lax.broadcasted_iota(jnp.int32, (1, _W), 1)

    def body(i, carry):
        sacc, iacc = carry
        Xv = X[...]
        m = jnp.max(Xv, axis=(1, 2), keepdims=True)
        cand = jnp.where(Xv == m, iota2, jnp.int32(1 << 30))
        idx = jnp.min(cand, axis=(1, 2), keepdims=True)
        X[...] = jnp.where(iota2 == idx, ninf, Xv)
        wmask = lane == i
        sacc = jnp.where(wmask, m.reshape(_NMAPS, 1), sacc)
        iacc = jnp.where(wmask, idx.reshape(_NMAPS, 1), iacc)
        return sacc, iacc

    scores, inds = jax.lax.fori_loop(
        0, _K, body,
        (jnp.zeros((_NMAPS, _W), f32),
         jnp.zeros((_NMAPS, _W), jnp.int32)))

    ind0 = inds[0:1, :]
    y0 = ind0 // _W
    x0 = ind0 % _W
    ysf = y0.astype(f32)
    xsf = x0.astype(f32)

    iota_h = jax.lax.broadcasted_iota(jnp.int32, (_H, _W), 0)
    rowselT = (iota_h == y0).astype(f32)
    stage1T = jnp.dot(featT_ref[...], rowselT,
                      preferred_element_type=f32)
    iota_cw = jax.lax.broadcasted_iota(jnp.int32, (_C * _W, _W), 0)
    xmask = (iota_cw % _W) == x0
    maskedT = jnp.where(xmask, stage1T, 0.0)
    sel_c = jax.lax.broadcasted_iota(jnp.int32, (_C, _C * _W), 1) // _W
    SelCT = (sel_c == jax.lax.broadcasted_iota(
        jnp.int32, (_C, _C * _W), 0)).astype(f32)
    gT = jnp.dot(SelCT, maskedT, preferred_element_type=f32)

    cidx = jax.lax.broadcasted_iota(jnp.int32, (_J, _C), 1)
    jidx = jax.lax.broadcasted_iota(jnp.int32, (_J, _C), 0)
    PxT = (cidx == 2 + 2 * jidx).astype(f32)
    PyT = (cidx == 3 + 2 * jidx).astype(f32)
    kx = jnp.dot(PxT, gT, preferred_element_type=f32) + xsf
    ky = jnp.dot(PyT, gT, preferred_element_type=f32) + ysf

    wh0 = gT[0:1, :]
    wh1 = gT[1:2, :]
    cx = xsf + gT[36:37, :]
    cy = ysf + gT[37:38, :]
    bl = cx - wh0 / 2
    bt = cy - wh1 / 2
    br = cx + wh0 / 2
    bb = cy + wh1 / 2

    sj = scores[1:_NMAPS, :]
    ij = inds[1:_NMAPS, :]
    hy_i = ij // _W
    hx_i = ij % _W
    hy = hy_i.astype(f32)
    hx = hx_i.astype(f32)

    iota_cw2 = jax.lax.broadcasted_iota(jnp.int32, (2 * _W, _W), 0)
    offx_rows = []
    offy_rows = []
    for j in range(_J):
        rs = (iota_h == hy_i[j:j + 1, :]).astype(f32)
        Aj = jnp.dot(hpT_ref[...], rs, preferred_element_type=f32)
        wm = (iota_cw2 % _W) == hx_i[j:j + 1, :]
        c0 = (iota_cw2 // _W) == 0
        c1 = (iota_cw2 // _W) == 1
        offx_rows.append(jnp.sum(jnp.where(wm & c0, Aj, 0.0),
                                 axis=0, keepdims=True))
        offy_rows.append(jnp.sum(jnp.where(wm & c1, Aj, 0.0),
                                 axis=0, keepdims=True))
    offx = jnp.concatenate(offx_rows, axis=0)
    offy = jnp.concatenate(offy_rows, axis=0)

    hx = hx + offx
    hy = hy + offy
    mv = sj > _THRESH
    hs_m = jnp.where(mv, sj, -1.0)
    hx_m = jnp.where(mv, hx, -10000.0)
    hy_m = jnp.where(mv, hy, -10000.0)

    dx = kx[:, :, None] - hx_m[:, None, :]
    dy = ky[:, :, None] - hy_m[:, None, :]
    dist = jnp.sqrt(dx * dx + dy * dy)
    cl = jax.lax.broadcasted_iota(jnp.int32, (1, 1, _W), 2)
    pinf = f32(jnp.inf)
    dist = jnp.where(cl < _K, dist, pinf)
    min_dist = jnp.min(dist, axis=2)
    icand = jnp.where(dist == min_dist[:, :, None], cl, jnp.int32(1 << 30))
    min_ind = jnp.min(icand, axis=2)

    onehot = cl == min_ind[:, :, None]
    hs_sel = jnp.sum(jnp.where(onehot, hs_m[:, None, :], 0.0), axis=2)
    hx_sel = jnp.sum(jnp.where(onehot, hx_m[:, None, :], 0.0), axis=2)
    hy_sel = jnp.sum(jnp.where(onehot, hy_m[:, None, :], 0.0), axis=2)

    use_orig = ((hx_sel < bl) | (hx_sel > br) | (hy_sel < bt) | (hy_sel > bb)
                | (hs_sel < _THRESH)
                | (min_dist > jnp.maximum(bb - bt, br - bl) * 0.3))
    fx = jnp.where(use_orig, kx, hx_sel)
    fy = jnp.where(use_orig, ky, hy_sel)

    out_ref[0:1, :] = bl
    out_ref[1:2, :] = bt
    out_ref[2:3, :] = br
    out_ref[3:4, :] = bb
    out_ref[4:5, :] = scores[0:1, :]
    out_ref[5:6, :] = jnp.zeros((1, _W), f32)
    out_ref[6:6 + _J, :] = fx
    out_ref[6 + _J:6 + 2 * _J, :] = fy


def kernel(hm, wh, hps, reg, hm_hp, hp_offset):
    f32 = jnp.float32
    heat = jnp.concatenate([hm[0], hm_hp[0]], axis=0)
    feat = jnp.concatenate([wh[0], hps[0], reg[0]], axis=0)
    featT = jnp.transpose(feat, (0, 2, 1)).reshape(_C * _W, _H)
    hpT = jnp.transpose(hp_offset[0], (0, 2, 1)).reshape(2 * _W, _H)

    out = pl.pallas_call(
        _decode_kernel,
        out_shape=jax.ShapeDtypeStruct((6 + 2 * _J, _W), f32),
        scratch_shapes=[
            pltpu.VMEM((_NMAPS, _H, _W), f32),
        ],
    )(heat, featT, hpT)

    kxT = out[6:6 + _J, :_K].T
    kyT = out[6 + _J:6 + 2 * _J, :_K].T
    kps = jnp.stack([kxT, kyT], axis=-1).reshape(_K, 2 * _J)
    dets = jnp.concatenate([
        out[0, :_K][:, None], out[1, :_K][:, None],
        out[2, :_K][:, None], out[3, :_K][:, None],
        out[4, :_K][:, None], kps, jnp.zeros((_K, 1), f32)], axis=1)
    return dets[None]


# speedup vs baseline: 8.6520x; 1.1328x over previous
import functools

import jax
import jax.numpy as jnp
from jax import lax
from jax.experimental import pallas as pl
from jax.experimental.pallas import tpu as pltpu
from jax.experimental.pallas import tpu_sc as plsc

_K = 100
_J = 17
_H = 128
_W = 128
_HW = _H * _W
_NMAPS = 1 + _J
_C = 38
_THRESH = 0.1
_NOUT = 112
_BIG = 1 << 30


def _nms_kernel(heat_ref, out_ref, l1_ref, l2_ref):
    f32 = jnp.float32
    ninf = f32(-jnp.inf)
    x = jax.nn.sigmoid(heat_ref[...])
    negc = jnp.full((_NMAPS, _H, 1), ninf, f32)
    xl = jnp.concatenate([x[:, :, 1:], negc], axis=2)
    xr = jnp.concatenate([negc, x[:, :, :-1]], axis=2)
    m1 = jnp.maximum(jnp.maximum(xl, xr), x)
    negr = jnp.full((_NMAPS, 1, _W), ninf, f32)
    mu = jnp.concatenate([m1[:, 1:, :], negr], axis=1)
    md = jnp.concatenate([negr, m1[:, :-1, :]], axis=1)
    hmax = jnp.maximum(jnp.maximum(mu, md), m1)
    nmsed = jnp.where(hmax == x, x, 0.0)
    out_ref[...] = nmsed
    l1 = jnp.max(nmsed.reshape(_NMAPS, _H, 8, 16), axis=3)
    l1_ref[...] = l1
    l2 = jnp.max(jnp.max(l1.reshape(_NMAPS, 64, 2, 8), axis=3), axis=2)
    l2_ref[...] = l2


def _topk_sc_kernel(heat_hbm, l1_hbm, l2_hbm, scores_hbm, inds_hbm,
                    vals, l1, l2, outs, outi):
    f32 = jnp.float32
    i32 = jnp.int32
    wid = lax.axis_index("s") * 2 + lax.axis_index("c")
    iota16 = lax.broadcasted_iota(i32, (16,), 0)

    def take16(v, idx):
        return lax.gather(
            v, idx[:, None],
            lax.GatherDimensionNumbers(offset_dims=(),
                                       collapsed_slice_dims=(0,),
                                       start_index_map=(0,)),
            (1,), mode=lax.GatherScatterMode.PROMISE_IN_BOUNDS)

    def bmax(v):
        for s in (8, 4, 2, 1):
            v = jnp.maximum(v, take16(v, iota16 ^ s))
        return v

    def bmin(v):
        for s in (8, 4, 2, 1):
            v = jnp.minimum(v, take16(v, iota16 ^ s))
        return v

    def to_scalar(vec):
        return vec[0]

    @pl.when(wid < _NMAPS)
    def _():
        pltpu.sync_copy(heat_hbm.at[pl.ds(wid * _HW, _HW)], vals)
        pltpu.sync_copy(l1_hbm.at[pl.ds(wid * 1024, 1024)], l1)
        pltpu.sync_copy(l2_hbm.at[pl.ds(wid * 64, 64)], l2)

        def extract(j, carry):
            ov, oi = carry
            a0 = l2[pl.ds(0, 16)]
            a1 = l2[pl.ds(16, 16)]
            a2 = l2[pl.ds(32, 16)]
            a3 = l2[pl.ds(48, 16)]
            mv = bmax(jnp.maximum(jnp.maximum(a0, a1),
                                  jnp.maximum(a2, a3)))
            s2v = bmin(jnp.minimum(
                jnp.minimum(jnp.where(a0 == mv, iota16, _BIG),
                            jnp.where(a1 == mv, iota16 + 16, _BIG)),
                jnp.minimum(jnp.where(a2 == mv, iota16 + 32, _BIG),
                            jnp.where(a3 == mv, iota16 + 48, _BIG))))
            s2 = to_scalar(s2v)
            c1 = l1[pl.ds(pl.multiple_of(s2 * 16, 16), 16)]
            s1v = bmin(jnp.where(c1 == mv, s2 * 16 + iota16, _BIG))
            s1 = to_scalar(s1v)
            vc = vals[pl.ds(pl.multiple_of(s1 * 16, 16), 16)]
            flatv = bmin(jnp.where(vc == mv, s1 * 16 + iota16, _BIG))

            vcn = jnp.where(s1 * 16 + iota16 == flatv, f32(-1.0), vc)
            vals[pl.ds(pl.multiple_of(s1 * 16, 16), 16)] = vcn
            nl1 = bmax(vcn)
            c1n = jnp.where(s2 * 16 + iota16 == s1v, nl1, c1)
            l1[pl.ds(pl.multiple_of(s2 * 16, 16), 16)] = c1n
            nl2 = bmax(c1n)
            base2 = pl.multiple_of((s2 // 16) * 16, 16)
            c2 = l2[pl.ds(base2, 16)]
            c2n = jnp.where(base2 + iota16 == s2v, nl2, c2)
            l2[pl.ds(base2, 16)] = c2n

            ov = jnp.where(iota16 == (j & 15), mv, ov)
            oi = jnp.where(iota16 == (j & 15), flatv, oi)

            @pl.when((j & 15) == 15)
            def _():
                outs[pl.ds(pl.multiple_of(j - 15, 16), 16)] = ov
                outi[pl.ds(pl.multiple_of(j - 15, 16), 16)] = oi

            return ov, oi

        lax.fori_loop(0, _NOUT, extract,
                      (jnp.zeros((16,), f32), jnp.zeros((16,), i32)))

        pltpu.sync_copy(outs, scores_hbm.at[pl.ds(wid * _W, _W)])
        pltpu.sync_copy(outi, inds_hbm.at[pl.ds(wid * _W, _W)])


@functools.partial(
    pl.kernel,
    mesh=plsc.VectorSubcoreMesh(core_axis_name="c", subcore_axis_name="s"),
    out_type=[jax.ShapeDtypeStruct((_NMAPS * _W,), jnp.float32),
              jax.ShapeDtypeStruct((_NMAPS * _W,), jnp.int32)],
    scratch_types=[
        pltpu.VMEM((_HW,), jnp.float32),
        pltpu.VMEM((1024,), jnp.float32),
        pltpu.VMEM((64,), jnp.float32),
        pltpu.VMEM((_W,), jnp.float32),
        pltpu.VMEM((_W,), jnp.int32),
    ],
)
def _topk_sc(heat_hbm, l1_hbm, l2_hbm, scores_hbm, inds_hbm,
             vals, l1, l2, outs, outi):
    _topk_sc_kernel(heat_hbm, l1_hbm, l2_hbm, scores_hbm, inds_hbm,
                    vals, l1, l2, outs, outi)


def _decode_kernel(scores_ref, inds_ref, featT_ref, hpT_ref, out_ref):
    f32 = jnp.float32
    scores = scores_ref[...]
    inds = inds_ref[...]

    ind0 = inds[0:1, :]
    y0 = ind0 // _W
    x0 = ind0 % _W
    ysf = y0.astype(f32)
    xsf = x0.astype(f32)

    iota_h = jax.lax.broadcasted_iota(jnp.int32, (_H, _W), 0)
    rowselT = (iota_h == y0).astype(f32)
    stage1T = jnp.dot(featT_ref[...], rowselT,
                      preferred_element_type=f32)
    iota_cw = jax.lax.broadcasted_iota(jnp.int32, (_C * _W, _W), 0)
    xmask = (iota_cw % _W) == x0
    maskedT = jnp.where(xmask, stage1T, 0.0)
    sel_c = jax.lax.broadcasted_iota(jnp.int32, (_C, _C * _W), 1) // _W
    SelCT = (sel_c == jax.lax.broadcasted_iota(
        jnp.int32, (_C, _C * _W), 0)).astype(f32)
    gT = jnp.dot(SelCT, maskedT, preferred_element_type=f32)

    cidx = jax.lax.broadcasted_iota(jnp.int32, (_J, _C), 1)
    jidx = jax.lax.broadcasted_iota(jnp.int32, (_J, _C), 0)
    PxT = (cidx == 2 + 2 * jidx).astype(f32)
    PyT = (cidx == 3 + 2 * jidx).astype(f32)
    kx = jnp.dot(PxT, gT, preferred_element_type=f32) + xsf
    ky = jnp.dot(PyT, gT, preferred_element_type=f32) + ysf

    wh0 = gT[0:1, :]
    wh1 = gT[1:2, :]
    cx = xsf + gT[36:37, :]
    cy = ysf + gT[37:38, :]
    bl = cx - wh0 / 2
    bt = cy - wh1 / 2
    br = cx + wh0 / 2
    bb = cy + wh1 / 2

    sj = scores[1:_NMAPS, :]
    ij = inds[1:_NMAPS, :]
    hy_i = ij // _W
    hx_i = ij % _W
    hy = hy_i.astype(f32)
    hx = hx_i.astype(f32)

    iota_cw2 = jax.lax.broadcasted_iota(jnp.int32, (2 * _W, _W), 0)
    offx_rows = []
    offy_rows = []
    for j in range(_J):
        rs = (iota_h == hy_i[j:j + 1, :]).astype(f32)
        Aj = jnp.dot(hpT_ref[...], rs, preferred_element_type=f32)
        wm = (iota_cw2 % _W) == hx_i[j:j + 1, :]
        c0 = (iota_cw2 // _W) == 0
        c1 = (iota_cw2 // _W) == 1
        offx_rows.append(jnp.sum(jnp.where(wm & c0, Aj, 0.0),
                                 axis=0, keepdims=True))
        offy_rows.append(jnp.sum(jnp.where(wm & c1, Aj, 0.0),
                                 axis=0, keepdims=True))
    offx = jnp.concatenate(offx_rows, axis=0)
    offy = jnp.concatenate(offy_rows, axis=0)

    hx = hx + offx
    hy = hy + offy
    mv = sj > _THRESH
    hs_m = jnp.where(mv, sj, -1.0)
    hx_m = jnp.where(mv, hx, -10000.0)
    hy_m = jnp.where(mv, hy, -10000.0)

    dx = kx[:, :, None] - hx_m[:, None, :]
    dy = ky[:, :, None] - hy_m[:, None, :]
    dist = jnp.sqrt(dx * dx + dy * dy)
    cl = jax.lax.broadcasted_iota(jnp.int32, (1, 1, _W), 2)
    dist = jnp.where(cl < _K, dist, jnp.float32(jnp.inf))
    min_dist = jnp.min(dist, axis=2)
    icand = jnp.where(dist == min_dist[:, :, None], cl, _BIG)
    min_ind = jnp.min(icand, axis=2)

    onehot = cl == min_ind[:, :, None]
    hs_sel = jnp.sum(jnp.where(onehot, hs_m[:, None, :], 0.0), axis=2)
    hx_sel = jnp.sum(jnp.where(onehot, hx_m[:, None, :], 0.0), axis=2)
    hy_sel = jnp.sum(jnp.where(onehot, hy_m[:, None, :], 0.0), axis=2)

    use_orig = ((hx_sel < bl) | (hx_sel > br) | (hy_sel < bt) | (hy_sel > bb)
                | (hs_sel < _THRESH)
                | (min_dist > jnp.maximum(bb - bt, br - bl) * 0.3))
    fx = jnp.where(use_orig, kx, hx_sel)
    fy = jnp.where(use_orig, ky, hy_sel)

    out_ref[0:1, :] = bl
    out_ref[1:2, :] = bt
    out_ref[2:3, :] = br
    out_ref[3:4, :] = bb
    out_ref[4:5, :] = scores[0:1, :]
    out_ref[5:6, :] = jnp.zeros((1, _W), f32)
    out_ref[6:6 + _J, :] = fx
    out_ref[6 + _J:6 + 2 * _J, :] = fy


def kernel(hm, wh, hps, reg, hm_hp, hp_offset):
    f32 = jnp.float32
    heat = jnp.concatenate([hm[0], hm_hp[0]], axis=0)
    feat = jnp.concatenate([wh[0], hps[0], reg[0]], axis=0)
    featT = jnp.transpose(feat, (0, 2, 1)).reshape(_C * _W, _H)
    hpT = jnp.transpose(hp_offset[0], (0, 2, 1)).reshape(2 * _W, _H)

    nmsed, l1, l2 = pl.pallas_call(
        _nms_kernel,
        out_shape=[jax.ShapeDtypeStruct((_NMAPS, _H, _W), f32),
                   jax.ShapeDtypeStruct((_NMAPS, _H, 8), f32),
                   jax.ShapeDtypeStruct((_NMAPS, 64), f32)],
    )(heat)

    scores_f, inds_f = _topk_sc(nmsed.reshape(_NMAPS * _HW),
                                l1.reshape(_NMAPS * 1024),
                                l2.reshape(_NMAPS * 64))
    scores = scores_f.reshape(_NMAPS, _W)
    inds = inds_f.reshape(_NMAPS, _W)

    out = pl.pallas_call(
        _decode_kernel,
        out_shape=jax.ShapeDtypeStruct((6 + 2 * _J, _W), f32),
    )(scores, inds, featT, hpT)

    kxT = out[6:6 + _J, :_K].T
    kyT = out[6 + _J:6 + 2 * _J, :_K].T
    kps = jnp.stack([kxT, kyT], axis=-1).reshape(_K, 2 * _J)
    dets = jnp.concatenate([
        out[0, :_K][:, None], out[1, :_K][:, None],
        out[2, :_K][:, None], out[3, :_K][:, None],
        out[4, :_K][:, None], kps, jnp.zeros((_K, 1), f32)], axis=1)
    return dets[None]


# speedup vs baseline: 9.3021x; 1.0751x over previous
import functools

import jax
import jax.numpy as jnp
from jax import lax
from jax.experimental import pallas as pl
from jax.experimental.pallas import tpu as pltpu
from jax.experimental.pallas import tpu_sc as plsc

_K = 100
_J = 17
_H = 128
_W = 128
_HW = _H * _W
_NMAPS = 1 + _J
_C = 38
_THRESH = 0.1
_NOUT = 112
_BIG = 1 << 30


def _nms_kernel(heat_ref, out_ref, l1_ref, l2_ref):
    f32 = jnp.float32
    ninf = f32(-jnp.inf)
    x = jax.nn.sigmoid(heat_ref[...])
    negc = jnp.full((_NMAPS, _H, 1), ninf, f32)
    xl = jnp.concatenate([x[:, :, 1:], negc], axis=2)
    xr = jnp.concatenate([negc, x[:, :, :-1]], axis=2)
    m1 = jnp.maximum(jnp.maximum(xl, xr), x)
    negr = jnp.full((_NMAPS, 1, _W), ninf, f32)
    mu = jnp.concatenate([m1[:, 1:, :], negr], axis=1)
    md = jnp.concatenate([negr, m1[:, :-1, :]], axis=1)
    hmax = jnp.maximum(jnp.maximum(mu, md), m1)
    nmsed = jnp.where(hmax == x, x, 0.0)
    out_ref[...] = nmsed
    l1 = jnp.max(nmsed.reshape(_NMAPS, _H, 8, 16), axis=3)
    l1_ref[...] = jnp.concatenate(
        [l1, jnp.full((_NMAPS, _H, 8), -1.0, f32)], axis=2)
    l2_ref[...] = jnp.max(nmsed, axis=2)


def _topk_sc_kernel(heat_hbm, l1_hbm, l2_hbm, scores_hbm, inds_hbm,
                    vals, l1, l2, outs, outi):
    f32 = jnp.float32
    i32 = jnp.int32
    wid = lax.axis_index("s") * 2 + lax.axis_index("c")
    iota16 = lax.broadcasted_iota(i32, (16,), 0)

    def take16(v, idx):
        return lax.gather(
            v, idx[:, None],
            lax.GatherDimensionNumbers(offset_dims=(),
                                       collapsed_slice_dims=(0,),
                                       start_index_map=(0,)),
            (1,), mode=lax.GatherScatterMode.PROMISE_IN_BOUNDS)

    def bmax(v):
        for s in (8, 4, 2, 1):
            v = jnp.maximum(v, take16(v, iota16 ^ s))
        return v

    def bmin(v):
        for s in (8, 4, 2, 1):
            v = jnp.minimum(v, take16(v, iota16 ^ s))
        return v

    def to_scalar(vec):
        return vec[0]

    @pl.when(wid < _NMAPS)
    def _():
        pltpu.sync_copy(heat_hbm.at[wid], vals)
        pltpu.sync_copy(l1_hbm.at[wid], l1)
        pltpu.sync_copy(l2_hbm.at[wid], l2)

        def extract(j, carry):
            ov, oi = carry
            rows = [l2[pl.ds(16 * q, 16)] for q in range(8)]
            t = rows[0]
            for q in range(1, 8):
                t = jnp.maximum(t, rows[q])
            mv = bmax(t)
            rsel = jnp.where(rows[0] == mv, iota16, _BIG)
            for q in range(1, 8):
                rsel = jnp.minimum(
                    rsel, jnp.where(rows[q] == mv, iota16 + 16 * q, _BIG))
            rv = bmin(rsel)
            r = to_scalar(rv)
            c1 = l1[r, pl.ds(0, 16)]
            qv = bmin(jnp.where(c1 == mv, iota16, _BIG))
            q = to_scalar(qv)
            vc = vals[r, pl.ds(pl.multiple_of(q * 16, 16), 16)]
            lv = bmin(jnp.where(vc == mv, iota16, _BIG))
            flatv = rv * _W + qv * 16 + lv

            vcn = jnp.where(iota16 == lv, f32(-1.0), vc)
            vals[r, pl.ds(pl.multiple_of(q * 16, 16), 16)] = vcn
            nl1 = bmax(vcn)
            c1n = jnp.where(iota16 == qv, nl1, c1)
            l1[r, pl.ds(0, 16)] = c1n
            nl2 = bmax(c1n)
            base2 = pl.multiple_of((r // 16) * 16, 16)
            c2 = l2[pl.ds(base2, 16)]
            c2n = jnp.where(base2 + iota16 == rv, nl2, c2)
            l2[pl.ds(base2, 16)] = c2n

            ov = jnp.where(iota16 == (j & 15), mv, ov)
            oi = jnp.where(iota16 == (j & 15), flatv, oi)

            @pl.when(((j & 15) == 15) | (j == _K - 1))
            def _():
                outs[pl.ds(pl.multiple_of((j // 16) * 16, 16), 16)] = ov
                outi[pl.ds(pl.multiple_of((j // 16) * 16, 16), 16)] = oi

            return ov, oi

        lax.fori_loop(0, _K, extract,
                      (jnp.zeros((16,), f32), jnp.zeros((16,), i32)))

        pltpu.sync_copy(outs, scores_hbm.at[wid])
        pltpu.sync_copy(outi, inds_hbm.at[wid])


@functools.partial(
    pl.kernel,
    mesh=plsc.VectorSubcoreMesh(core_axis_name="c", subcore_axis_name="s"),
    out_type=[jax.ShapeDtypeStruct((_NMAPS, _W), jnp.float32),
              jax.ShapeDtypeStruct((_NMAPS, _W), jnp.int32)],
    scratch_types=[
        pltpu.VMEM((_H, _W), jnp.float32),
        pltpu.VMEM((_H, 16), jnp.float32),
        pltpu.VMEM((_H,), jnp.float32),
        pltpu.VMEM((_W,), jnp.float32),
        pltpu.VMEM((_W,), jnp.int32),
    ],
)
def _topk_sc(heat_hbm, l1_hbm, l2_hbm, scores_hbm, inds_hbm,
             vals, l1, l2, outs, outi):
    _topk_sc_kernel(heat_hbm, l1_hbm, l2_hbm, scores_hbm, inds_hbm,
                    vals, l1, l2, outs, outi)


def _decode_kernel(scores_ref, inds_ref, featT_ref, hpT_ref, out_ref):
    f32 = jnp.float32
    scores = scores_ref[...]
    inds = inds_ref[...]

    ind0 = inds[0:1, :]
    y0 = ind0 // _W
    x0 = ind0 % _W
    ysf = y0.astype(f32)
    xsf = x0.astype(f32)

    iota_h = jax.lax.broadcasted_iota(jnp.int32, (_H, _W), 0)
    rowselT = (iota_h == y0).astype(f32)
    stage1T = jnp.dot(featT_ref[...], rowselT,
                      preferred_element_type=f32)
    iota_cw = jax.lax.broadcasted_iota(jnp.int32, (_C * _W, _W), 0)
    xmask = (iota_cw % _W) == x0
    maskedT = jnp.where(xmask, stage1T, 0.0)
    sel_c = jax.lax.broadcasted_iota(jnp.int32, (_C, _C * _W), 1) // _W
    SelCT = (sel_c == jax.lax.broadcasted_iota(
        jnp.int32, (_C, _C * _W), 0)).astype(f32)
    gT = jnp.dot(SelCT, maskedT, preferred_element_type=f32)

    cidx = jax.lax.broadcasted_iota(jnp.int32, (_J, _C), 1)
    jidx = jax.lax.broadcasted_iota(jnp.int32, (_J, _C), 0)
    PxT = (cidx == 2 + 2 * jidx).astype(f32)
    PyT = (cidx == 3 + 2 * jidx).astype(f32)
    kx = jnp.dot(PxT, gT, preferred_element_type=f32) + xsf
    ky = jnp.dot(PyT, gT, preferred_element_type=f32) + ysf

    wh0 = gT[0:1, :]
    wh1 = gT[1:2, :]
    cx = xsf + gT[36:37, :]
    cy = ysf + gT[37:38, :]
    bl = cx - wh0 / 2
    bt = cy - wh1 / 2
    br = cx + wh0 / 2
    bb = cy + wh1 / 2

    sj = scores[1:_NMAPS, :]
    ij = inds[1:_NMAPS, :]
    hy_i = ij // _W
    hx_i = ij % _W
    hy = hy_i.astype(f32)
    hx = hx_i.astype(f32)

    iota_cw2 = jax.lax.broadcasted_iota(jnp.int32, (2 * _W, _W), 0)
    offx_rows = []
    offy_rows = []
    for j in range(_J):
        rs = (iota_h == hy_i[j:j + 1, :]).astype(f32)
        Aj = jnp.dot(hpT_ref[...], rs, preferred_element_type=f32)
        wm = (iota_cw2 % _W) == hx_i[j:j + 1, :]
        c0 = (iota_cw2 // _W) == 0
        c1 = (iota_cw2 // _W) == 1
        offx_rows.append(jnp.sum(jnp.where(wm & c0, Aj, 0.0),
                                 axis=0, keepdims=True))
        offy_rows.append(jnp.sum(jnp.where(wm & c1, Aj, 0.0),
                                 axis=0, keepdims=True))
    offx = jnp.concatenate(offx_rows, axis=0)
    offy = jnp.concatenate(offy_rows, axis=0)

    hx = hx + offx
    hy = hy + offy
    mv = sj > _THRESH
    hs_m = jnp.where(mv, sj, -1.0)
    hx_m = jnp.where(mv, hx, -10000.0)
    hy_m = jnp.where(mv, hy, -10000.0)

    dx = kx[:, :, None] - hx_m[:, None, :]
    dy = ky[:, :, None] - hy_m[:, None, :]
    dist = jnp.sqrt(dx * dx + dy * dy)
    cl = jax.lax.broadcasted_iota(jnp.int32, (1, 1, _W), 2)
    dist = jnp.where(cl < _K, dist, jnp.float32(jnp.inf))
    min_dist = jnp.min(dist, axis=2)
    icand = jnp.where(dist == min_dist[:, :, None], cl, _BIG)
    min_ind = jnp.min(icand, axis=2)

    onehot = cl == min_ind[:, :, None]
    hs_sel = jnp.sum(jnp.where(onehot, hs_m[:, None, :], 0.0), axis=2)
    hx_sel = jnp.sum(jnp.where(onehot, hx_m[:, None, :], 0.0), axis=2)
    hy_sel = jnp.sum(jnp.where(onehot, hy_m[:, None, :], 0.0), axis=2)

    use_orig = ((hx_sel < bl) | (hx_sel > br) | (hy_sel < bt) | (hy_sel > bb)
                | (hs_sel < _THRESH)
                | (min_dist > jnp.maximum(bb - bt, br - bl) * 0.3))
    fx = jnp.where(use_orig, kx, hx_sel)
    fy = jnp.where(use_orig, ky, hy_sel)

    out_ref[0:1, :] = bl
    out_ref[1:2, :] = bt
    out_ref[2:3, :] = br
    out_ref[3:4, :] = bb
    out_ref[4:5, :] = scores[0:1, :]
    out_ref[5:6, :] = jnp.zeros((1, _W), f32)
    out_ref[6:6 + _J, :] = fx
    out_ref[6 + _J:6 + 2 * _J, :] = fy


def kernel(hm, wh, hps, reg, hm_hp, hp_offset):
    f32 = jnp.float32
    heat = jnp.concatenate([hm[0], hm_hp[0]], axis=0)
    feat = jnp.concatenate([wh[0], hps[0], reg[0]], axis=0)
    featT = jnp.transpose(feat, (0, 2, 1)).reshape(_C * _W, _H)
    hpT = jnp.transpose(hp_offset[0], (0, 2, 1)).reshape(2 * _W, _H)

    nmsed, l1, l2 = pl.pallas_call(
        _nms_kernel,
        out_shape=[jax.ShapeDtypeStruct((_NMAPS, _H, _W), f32),
                   jax.ShapeDtypeStruct((_NMAPS, _H, 16), f32),
                   jax.ShapeDtypeStruct((_NMAPS, _H), f32)],
    )(heat)

    scores, inds = _topk_sc(nmsed, l1, l2)

    out = pl.pallas_call(
        _decode_kernel,
        out_shape=jax.ShapeDtypeStruct((6 + 2 * _J, _W), f32),
    )(scores, inds, featT, hpT)

    kxT = out[6:6 + _J, :_K].T
    kyT = out[6 + _J:6 + 2 * _J, :_K].T
    kps = jnp.stack([kxT, kyT], axis=-1).reshape(_K, 2 * _J)
    dets = jnp.concatenate([
        out[0, :_K][:, None], out[1, :_K][:, None],
        out[2, :_K][:, None], out[3, :_K][:, None],
        out[4, :_K][:, None], kps, jnp.zeros((_K, 1), f32)], axis=1)
    return dets[None]


# speedup vs baseline: 10.4400x; 1.1223x over previous
import functools

import jax
import jax.numpy as jnp
from jax import lax
from jax.experimental import pallas as pl
from jax.experimental.pallas import tpu as pltpu
from jax.experimental.pallas import tpu_sc as plsc

_K = 100
_J = 17
_H = 128
_W = 128
_HW = _H * _W
_NMAPS = 1 + _J
_C = 38
_THRESH = 0.1
_NOUT = 112
_BIG = 1 << 30


def _nms_kernel(hm_ref, hmhp_ref, out_ref, l1_ref, l2_ref):
    f32 = jnp.float32
    ninf = f32(-jnp.inf)
    x = jax.nn.sigmoid(jnp.concatenate(
        [hm_ref[...], hmhp_ref[...]], axis=0))
    negc = jnp.full((_NMAPS, _H, 1), ninf, f32)
    xl = jnp.concatenate([x[:, :, 1:], negc], axis=2)
    xr = jnp.concatenate([negc, x[:, :, :-1]], axis=2)
    m1 = jnp.maximum(jnp.maximum(xl, xr), x)
    negr = jnp.full((_NMAPS, 1, _W), ninf, f32)
    mu = jnp.concatenate([m1[:, 1:, :], negr], axis=1)
    md = jnp.concatenate([negr, m1[:, :-1, :]], axis=1)
    hmax = jnp.maximum(jnp.maximum(mu, md), m1)
    nmsed = jnp.where(hmax == x, x, 0.0)
    out_ref[...] = nmsed
    l1 = jnp.max(nmsed.reshape(_NMAPS, _H, 8, 16), axis=3)
    l1_ref[...] = jnp.concatenate(
        [l1, jnp.full((_NMAPS, _H, 8), -1.0, f32)], axis=2)
    l2_ref[...] = jnp.max(nmsed, axis=2)


def _topk_sc_kernel(heat_hbm, l1_hbm, l2_hbm, scores_hbm, inds_hbm,
                    vals, l1, l2, outs, outi):
    f32 = jnp.float32
    i32 = jnp.int32
    wid = lax.axis_index("s") * 2 + lax.axis_index("c")
    iota16 = lax.broadcasted_iota(i32, (16,), 0)

    def take16(v, idx):
        return lax.gather(
            v, idx[:, None],
            lax.GatherDimensionNumbers(offset_dims=(),
                                       collapsed_slice_dims=(0,),
                                       start_index_map=(0,)),
            (1,), mode=lax.GatherScatterMode.PROMISE_IN_BOUNDS)

    def bmax(v):
        for s in (8, 4, 2, 1):
            v = jnp.maximum(v, take16(v, iota16 ^ s))
        return v

    def bmin(v):
        for s in (8, 4, 2, 1):
            v = jnp.minimum(v, take16(v, iota16 ^ s))
        return v

    def to_scalar(vec):
        return vec[0]

    @pl.when(wid < _NMAPS)
    def _():
        pltpu.sync_copy(heat_hbm.at[wid], vals)
        pltpu.sync_copy(l1_hbm.at[wid], l1)
        pltpu.sync_copy(l2_hbm.at[wid], l2)

        def extract(j, carry):
            ov, oi = carry
            rows = [l2[pl.ds(16 * q, 16)] for q in range(8)]
            t = rows[0]
            for q in range(1, 8):
                t = jnp.maximum(t, rows[q])
            mv = bmax(t)
            rsel = jnp.where(rows[0] == mv, iota16, _BIG)
            for q in range(1, 8):
                rsel = jnp.minimum(
                    rsel, jnp.where(rows[q] == mv, iota16 + 16 * q, _BIG))
            rv = bmin(rsel)
            r = to_scalar(rv)
            c1 = l1[r, pl.ds(0, 16)]
            qv = bmin(jnp.where(c1 == mv, iota16, _BIG))
            q = to_scalar(qv)
            vc = vals[r, pl.ds(pl.multiple_of(q * 16, 16), 16)]
            lv = bmin(jnp.where(vc == mv, iota16, _BIG))
            flatv = rv * _W + qv * 16 + lv

            vcn = jnp.where(iota16 == lv, f32(-1.0), vc)
            vals[r, pl.ds(pl.multiple_of(q * 16, 16), 16)] = vcn
            nl1 = bmax(vcn)
            c1n = jnp.where(iota16 == qv, nl1, c1)
            l1[r, pl.ds(0, 16)] = c1n
            nl2 = bmax(c1n)
            base2 = pl.multiple_of((r // 16) * 16, 16)
            c2 = l2[pl.ds(base2, 16)]
            c2n = jnp.where(base2 + iota16 == rv, nl2, c2)
            l2[pl.ds(base2, 16)] = c2n

            ov = jnp.where(iota16 == (j & 15), mv, ov)
            oi = jnp.where(iota16 == (j & 15), flatv, oi)

            @pl.when(((j & 15) == 15) | (j == _K - 1))
            def _():
                outs[pl.ds(pl.multiple_of((j // 16) * 16, 16), 16)] = ov
                outi[pl.ds(pl.multiple_of((j // 16) * 16, 16), 16)] = oi

            return ov, oi

        lax.fori_loop(0, _K, extract,
                      (jnp.zeros((16,), f32), jnp.zeros((16,), i32)))

        pltpu.sync_copy(outs, scores_hbm.at[wid])
        pltpu.sync_copy(outi, inds_hbm.at[wid])


@functools.partial(
    pl.kernel,
    mesh=plsc.VectorSubcoreMesh(core_axis_name="c", subcore_axis_name="s"),
    out_type=[jax.ShapeDtypeStruct((_NMAPS, _W), jnp.float32),
              jax.ShapeDtypeStruct((_NMAPS, _W), jnp.int32)],
    scratch_types=[
        pltpu.VMEM((_H, _W), jnp.float32),
        pltpu.VMEM((_H, 16), jnp.float32),
        pltpu.VMEM((_H,), jnp.float32),
        pltpu.VMEM((_W,), jnp.float32),
        pltpu.VMEM((_W,), jnp.int32),
    ],
)
def _topk_sc(heat_hbm, l1_hbm, l2_hbm, scores_hbm, inds_hbm,
             vals, l1, l2, outs, outi):
    _topk_sc_kernel(heat_hbm, l1_hbm, l2_hbm, scores_hbm, inds_hbm,
                    vals, l1, l2, outs, outi)


def _decode_kernel(scores_ref, inds_ref, featT_ref, hpT_ref, out_ref):
    f32 = jnp.float32
    scores = scores_ref[...]
    inds = inds_ref[...]

    ind0 = inds[0:1, :]
    y0 = ind0 // _W
    x0 = ind0 % _W
    ysf = y0.astype(f32)
    xsf = x0.astype(f32)

    iota_h = jax.lax.broadcasted_iota(jnp.int32, (_H, _W), 0)
    rowselT = (iota_h == y0).astype(f32)
    stage1T = jnp.dot(featT_ref[...], rowselT,
                      preferred_element_type=f32)
    iota_cw = jax.lax.broadcasted_iota(jnp.int32, (_C * _W, _W), 0)
    xmask = (iota_cw % _W) == x0
    maskedT = jnp.where(xmask, stage1T, 0.0)
    sel_c = jax.lax.broadcasted_iota(jnp.int32, (_C, _C * _W), 1) // _W
    SelCT = (sel_c == jax.lax.broadcasted_iota(
        jnp.int32, (_C, _C * _W), 0)).astype(f32)
    gT = jnp.dot(SelCT, maskedT, preferred_element_type=f32)

    cidx = jax.lax.broadcasted_iota(jnp.int32, (_J, _C), 1)
    jidx = jax.lax.broadcasted_iota(jnp.int32, (_J, _C), 0)
    PxT = (cidx == 2 + 2 * jidx).astype(f32)
    PyT = (cidx == 3 + 2 * jidx).astype(f32)
    kx = jnp.dot(PxT, gT, preferred_element_type=f32) + xsf
    ky = jnp.dot(PyT, gT, preferred_element_type=f32) + ysf

    wh0 = gT[0:1, :]
    wh1 = gT[1:2, :]
    cx = xsf + gT[36:37, :]
    cy = ysf + gT[37:38, :]
    bl = cx - wh0 / 2
    bt = cy - wh1 / 2
    br = cx + wh0 / 2
    bb = cy + wh1 / 2

    sj = scores[1:_NMAPS, :]
    ij = inds[1:_NMAPS, :]
    hy_i = ij // _W
    hx_i = ij % _W
    hy = hy_i.astype(f32)
    hx = hx_i.astype(f32)

    iota_cw2 = jax.lax.broadcasted_iota(jnp.int32, (2 * _W, _W), 0)
    offx_rows = []
    offy_rows = []
    for j in range(_J):
        rs = (iota_h == hy_i[j:j + 1, :]).astype(f32)
        Aj = jnp.dot(hpT_ref[...], rs, preferred_element_type=f32)
        wm = (iota_cw2 % _W) == hx_i[j:j + 1, :]
        c0 = (iota_cw2 // _W) == 0
        c1 = (iota_cw2 // _W) == 1
        offx_rows.append(jnp.sum(jnp.where(wm & c0, Aj, 0.0),
                                 axis=0, keepdims=True))
        offy_rows.append(jnp.sum(jnp.where(wm & c1, Aj, 0.0),
                                 axis=0, keepdims=True))
    offx = jnp.concatenate(offx_rows, axis=0)
    offy = jnp.concatenate(offy_rows, axis=0)

    hx = hx + offx
    hy = hy + offy
    mv = sj > _THRESH
    hs_m = jnp.where(mv, sj, -1.0)
    hx_m = jnp.where(mv, hx, -10000.0)
    hy_m = jnp.where(mv, hy, -10000.0)

    dx = kx[:, :, None] - hx_m[:, None, :]
    dy = ky[:, :, None] - hy_m[:, None, :]
    dist = jnp.sqrt(dx * dx + dy * dy)
    cl = jax.lax.broadcasted_iota(jnp.int32, (1, 1, _W), 2)
    dist = jnp.where(cl < _K, dist, jnp.float32(jnp.inf))
    min_dist = jnp.min(dist, axis=2)
    icand = jnp.where(dist == min_dist[:, :, None], cl, _BIG)
    min_ind = jnp.min(icand, axis=2)

    onehot = cl == min_ind[:, :, None]
    hs_sel = jnp.sum(jnp.where(onehot, hs_m[:, None, :], 0.0), axis=2)
    hx_sel = jnp.sum(jnp.where(onehot, hx_m[:, None, :], 0.0), axis=2)
    hy_sel = jnp.sum(jnp.where(onehot, hy_m[:, None, :], 0.0), axis=2)

    use_orig = ((hx_sel < bl) | (hx_sel > br) | (hy_sel < bt) | (hy_sel > bb)
                | (hs_sel < _THRESH)
                | (min_dist > jnp.maximum(bb - bt, br - bl) * 0.3))
    fx = jnp.where(use_orig, kx, hx_sel)
    fy = jnp.where(use_orig, ky, hy_sel)

    rows = [bl, bt, br, bb, scores[0:1, :]]
    for t in range(_J):
        rows.append(fx[t:t + 1, :])
        rows.append(fy[t:t + 1, :])
    rows.append(jnp.zeros((1, _W), f32))
    outT = jnp.concatenate(rows, axis=0)
    out_ref[...] = outT.T


def kernel(hm, wh, hps, reg, hm_hp, hp_offset):
    f32 = jnp.float32
    feat = jnp.concatenate([wh[0], hps[0], reg[0]], axis=0)
    featT = jnp.transpose(feat, (0, 2, 1)).reshape(_C * _W, _H)
    hpT = jnp.transpose(hp_offset[0], (0, 2, 1)).reshape(2 * _W, _H)

    nmsed, l1, l2 = pl.pallas_call(
        _nms_kernel,
        out_shape=[jax.ShapeDtypeStruct((_NMAPS, _H, _W), f32),
                   jax.ShapeDtypeStruct((_NMAPS, _H, 16), f32),
                   jax.ShapeDtypeStruct((_NMAPS, _H), f32)],
    )(hm[0], hm_hp[0])

    scores, inds = _topk_sc(nmsed, l1, l2)

    out = pl.pallas_call(
        _decode_kernel,
        out_shape=jax.ShapeDtypeStruct((_H, 40), f32),
    )(scores, inds, featT, hpT)

    return out[:_K][None]


# speedup vs baseline: 10.6709x; 1.0221x over previous
import functools

import jax
import jax.numpy as jnp
from jax import lax
from jax.experimental import pallas as pl
from jax.experimental.pallas import tpu as pltpu
from jax.experimental.pallas import tpu_sc as plsc

_K = 100
_J = 17
_H = 128
_W = 128
_HW = _H * _W
_NMAPS = 1 + _J
_C = 38
_THRESH = 0.1
_NOUT = 112
_BIG = 1 << 30


def _nms_kernel(hm_ref, hmhp_ref, out_ref, l1_ref, l2_ref):
    f32 = jnp.float32
    ninf = f32(-jnp.inf)
    x = jax.nn.sigmoid(jnp.concatenate(
        [hm_ref[...], hmhp_ref[...]], axis=0))
    negc = jnp.full((_NMAPS, _H, 1), ninf, f32)
    xl = jnp.concatenate([x[:, :, 1:], negc], axis=2)
    xr = jnp.concatenate([negc, x[:, :, :-1]], axis=2)
    m1 = jnp.maximum(jnp.maximum(xl, xr), x)
    negr = jnp.full((_NMAPS, 1, _W), ninf, f32)
    mu = jnp.concatenate([m1[:, 1:, :], negr], axis=1)
    md = jnp.concatenate([negr, m1[:, :-1, :]], axis=1)
    hmax = jnp.maximum(jnp.maximum(mu, md), m1)
    nmsed = jnp.where(hmax == x, x, 0.0)
    out_ref[...] = nmsed
    l1 = jnp.max(nmsed.reshape(_NMAPS, _H, 8, 16), axis=3)
    l1_ref[...] = jnp.concatenate(
        [l1, jnp.full((_NMAPS, _H, 8), -1.0, f32)], axis=2)
    l2_ref[...] = jnp.max(nmsed, axis=2)


def _topk_sc_kernel(heat_hbm, l1_hbm, l2_hbm, scores_hbm, inds_hbm,
                    vals, l1, l2, outs, outi):
    f32 = jnp.float32
    i32 = jnp.int32
    wid = lax.axis_index("s") * 2 + lax.axis_index("c")
    iota16 = lax.broadcasted_iota(i32, (16,), 0)

    def take16(v, idx):
        return lax.gather(
            v, idx[:, None],
            lax.GatherDimensionNumbers(offset_dims=(),
                                       collapsed_slice_dims=(0,),
                                       start_index_map=(0,)),
            (1,), mode=lax.GatherScatterMode.PROMISE_IN_BOUNDS)

    def bmax(v):
        for s in (8, 4, 2, 1):
            v = jnp.maximum(v, take16(v, iota16 ^ s))
        return v

    def bmin(v):
        for s in (8, 4, 2, 1):
            v = jnp.minimum(v, take16(v, iota16 ^ s))
        return v

    def to_scalar(vec):
        return vec[0]

    @pl.when(wid < _NMAPS)
    def _():
        pltpu.sync_copy(heat_hbm.at[wid], vals)
        pltpu.sync_copy(l1_hbm.at[wid], l1)
        pltpu.sync_copy(l2_hbm.at[wid], l2)

        def extract(j, carry):
            ov, oi = carry
            rows = [l2[pl.ds(16 * q, 16)] for q in range(8)]
            t = rows[0]
            for q in range(1, 8):
                t = jnp.maximum(t, rows[q])
            mv = bmax(t)
            rsel = jnp.where(rows[0] == mv, iota16, _BIG)
            for q in range(1, 8):
                rsel = jnp.minimum(
                    rsel, jnp.where(rows[q] == mv, iota16 + 16 * q, _BIG))
            rv = bmin(rsel)
            r = to_scalar(rv)
            c1 = l1[r, pl.ds(0, 16)]
            qv = bmin(jnp.where(c1 == mv, iota16, _BIG))
            q = to_scalar(qv)
            vc = vals[r, pl.ds(pl.multiple_of(q * 16, 16), 16)]
            lv = bmin(jnp.where(vc == mv, iota16, _BIG))
            flatv = rv * _W + qv * 16 + lv

            vcn = jnp.where(iota16 == lv, f32(-1.0), vc)
            vals[r, pl.ds(pl.multiple_of(q * 16, 16), 16)] = vcn
            nl1 = bmax(vcn)
            c1n = jnp.where(iota16 == qv, nl1, c1)
            l1[r, pl.ds(0, 16)] = c1n
            nl2 = bmax(c1n)
            base2 = pl.multiple_of((r // 16) * 16, 16)
            c2 = l2[pl.ds(base2, 16)]
            c2n = jnp.where(base2 + iota16 == rv, nl2, c2)
            l2[pl.ds(base2, 16)] = c2n

            ov = jnp.where(iota16 == (j & 15), mv, ov)
            oi = jnp.where(iota16 == (j & 15), flatv, oi)

            @pl.when(((j & 15) == 15) | (j == _K - 1))
            def _():
                outs[pl.ds(pl.multiple_of((j // 16) * 16, 16), 16)] = ov
                outi[pl.ds(pl.multiple_of((j // 16) * 16, 16), 16)] = oi

            return ov, oi

        lax.fori_loop(0, _K, extract,
                      (jnp.zeros((16,), f32), jnp.zeros((16,), i32)))

        pltpu.sync_copy(outs, scores_hbm.at[wid])
        pltpu.sync_copy(outi, inds_hbm.at[wid])


@functools.partial(
    pl.kernel,
    mesh=plsc.VectorSubcoreMesh(core_axis_name="c", subcore_axis_name="s"),
    out_type=[jax.ShapeDtypeStruct((_NMAPS, _W), jnp.float32),
              jax.ShapeDtypeStruct((_NMAPS, _W), jnp.int32)],
    scratch_types=[
        pltpu.VMEM((_H, _W), jnp.float32),
        pltpu.VMEM((_H, 16), jnp.float32),
        pltpu.VMEM((_H,), jnp.float32),
        pltpu.VMEM((_W,), jnp.float32),
        pltpu.VMEM((_W,), jnp.int32),
    ],
)
def _topk_sc(heat_hbm, l1_hbm, l2_hbm, scores_hbm, inds_hbm,
             vals, l1, l2, outs, outi):
    _topk_sc_kernel(heat_hbm, l1_hbm, l2_hbm, scores_hbm, inds_hbm,
                    vals, l1, l2, outs, outi)


def _decode_kernel(scores_ref, inds_ref, wh_ref, hps_ref, reg_ref, hp_ref,
                   out_ref):
    f32 = jnp.float32
    scores = scores_ref[...]
    inds = inds_ref[...]
    feat = jnp.concatenate([wh_ref[...], hps_ref[...], reg_ref[...]], axis=0)
    featT = jnp.transpose(feat, (0, 2, 1)).reshape(_C * _W, _H)
    hpT = jnp.transpose(hp_ref[...], (0, 2, 1)).reshape(2 * _W, _H)

    ind0 = inds[0:1, :]
    y0 = ind0 // _W
    x0 = ind0 % _W
    ysf = y0.astype(f32)
    xsf = x0.astype(f32)

    iota_h = jax.lax.broadcasted_iota(jnp.int32, (_H, _W), 0)
    rowselT = (iota_h == y0).astype(f32)
    stage1T = jnp.dot(featT, rowselT,
                      preferred_element_type=f32)
    iota_cw = jax.lax.broadcasted_iota(jnp.int32, (_C * _W, _W), 0)
    xmask = (iota_cw % _W) == x0
    maskedT = jnp.where(xmask, stage1T, 0.0)
    sel_c = jax.lax.broadcasted_iota(jnp.int32, (_C, _C * _W), 1) // _W
    SelCT = (sel_c == jax.lax.broadcasted_iota(
        jnp.int32, (_C, _C * _W), 0)).astype(f32)
    gT = jnp.dot(SelCT, maskedT, preferred_element_type=f32)

    cidx = jax.lax.broadcasted_iota(jnp.int32, (_J, _C), 1)
    jidx = jax.lax.broadcasted_iota(jnp.int32, (_J, _C), 0)
    PxT = (cidx == 2 + 2 * jidx).astype(f32)
    PyT = (cidx == 3 + 2 * jidx).astype(f32)
    kx = jnp.dot(PxT, gT, preferred_element_type=f32) + xsf
    ky = jnp.dot(PyT, gT, preferred_element_type=f32) + ysf

    wh0 = gT[0:1, :]
    wh1 = gT[1:2, :]
    cx = xsf + gT[36:37, :]
    cy = ysf + gT[37:38, :]
    bl = cx - wh0 / 2
    bt = cy - wh1 / 2
    br = cx + wh0 / 2
    bb = cy + wh1 / 2

    sj = scores[1:_NMAPS, :]
    ij = inds[1:_NMAPS, :]
    hy_i = ij // _W
    hx_i = ij % _W
    hy = hy_i.astype(f32)
    hx = hx_i.astype(f32)

    iota_cw2 = jax.lax.broadcasted_iota(jnp.int32, (2 * _W, _W), 0)
    offx_rows = []
    offy_rows = []
    for j in range(_J):
        rs = (iota_h == hy_i[j:j + 1, :]).astype(f32)
        Aj = jnp.dot(hpT, rs, preferred_element_type=f32)
        wm = (iota_cw2 % _W) == hx_i[j:j + 1, :]
        c0 = (iota_cw2 // _W) == 0
        c1 = (iota_cw2 // _W) == 1
        offx_rows.append(jnp.sum(jnp.where(wm & c0, Aj, 0.0),
                                 axis=0, keepdims=True))
        offy_rows.append(jnp.sum(jnp.where(wm & c1, Aj, 0.0),
                                 axis=0, keepdims=True))
    offx = jnp.concatenate(offx_rows, axis=0)
    offy = jnp.concatenate(offy_rows, axis=0)

    hx = hx + offx
    hy = hy + offy
    mv = sj > _THRESH
    hs_m = jnp.where(mv, sj, -1.0)
    hx_m = jnp.where(mv, hx, -10000.0)
    hy_m = jnp.where(mv, hy, -10000.0)

    dx = kx[:, :, None] - hx_m[:, None, :]
    dy = ky[:, :, None] - hy_m[:, None, :]
    dist = jnp.sqrt(dx * dx + dy * dy)
    cl = jax.lax.broadcasted_iota(jnp.int32, (1, 1, _W), 2)
    dist = jnp.where(cl < _K, dist, jnp.float32(jnp.inf))
    min_dist = jnp.min(dist, axis=2)
    icand = jnp.where(dist == min_dist[:, :, None], cl, _BIG)
    min_ind = jnp.min(icand, axis=2)

    onehot = cl == min_ind[:, :, None]
    hs_sel = jnp.sum(jnp.where(onehot, hs_m[:, None, :], 0.0), axis=2)
    hx_sel = jnp.sum(jnp.where(onehot, hx_m[:, None, :], 0.0), axis=2)
    hy_sel = jnp.sum(jnp.where(onehot, hy_m[:, None, :], 0.0), axis=2)

    use_orig = ((hx_sel < bl) | (hx_sel > br) | (hy_sel < bt) | (hy_sel > bb)
                | (hs_sel < _THRESH)
                | (min_dist > jnp.maximum(bb - bt, br - bl) * 0.3))
    fx = jnp.where(use_orig, kx, hx_sel)
    fy = jnp.where(use_orig, ky, hy_sel)

    rows = [bl, bt, br, bb, scores[0:1, :]]
    for t in range(_J):
        rows.append(fx[t:t + 1, :])
        rows.append(fy[t:t + 1, :])
    rows.append(jnp.zeros((1, _W), f32))
    outT = jnp.concatenate(rows, axis=0)
    out_ref[...] = outT.T


def kernel(hm, wh, hps, reg, hm_hp, hp_offset):
    f32 = jnp.float32
    nmsed, l1, l2 = pl.pallas_call(
        _nms_kernel,
        out_shape=[jax.ShapeDtypeStruct((_NMAPS, _H, _W), f32),
                   jax.ShapeDtypeStruct((_NMAPS, _H, 16), f32),
                   jax.ShapeDtypeStruct((_NMAPS, _H), f32)],
    )(hm[0], hm_hp[0])

    scores, inds = _topk_sc(nmsed, l1, l2)

    out = pl.pallas_call(
        _decode_kernel,
        out_shape=jax.ShapeDtypeStruct((_H, 40), f32),
    )(scores, inds, wh[0], hps[0], reg[0], hp_offset[0])

    return out[:_K][None]
